# baseline (speedup 1.0000x reference)
import functools

import jax
import jax.numpy as jnp
from jax import lax
from jax.experimental import pallas as pl
from jax.experimental.pallas import tpu as pltpu
from jax.experimental.pallas import tpu_sc as plsc

_EMBED = 512
_NC = 2
_NS = 16
_NW = _NC * _NS
_NBUF = 4
_SEQP = 56


@functools.cache
def _build(n_batch: int, seq: int):
    bat_per_w = n_batch // _NW
    n_groups = bat_per_w // _NBUF
    assert bat_per_w * _NW == n_batch
    assert n_groups * _NBUF == bat_per_w

    mesh = plsc.VectorSubcoreMesh(
        core_axis_name="c", subcore_axis_name="s", num_cores=_NC, num_subcores=_NS
    )

    @functools.partial(
        pl.kernel,
        mesh=mesh,
        out_type=jax.ShapeDtypeStruct((n_batch, seq, _EMBED), jnp.float32),
        scratch_types=[
            pltpu.VMEM((bat_per_w * _SEQP,), jnp.int32),
            pltpu.VMEM((_NBUF, _SEQP, _EMBED), jnp.float32),
            pltpu.SemaphoreType.DMA,
            pltpu.SemaphoreType.DMA,
            pltpu.SemaphoreType.DMA,
            pltpu.SemaphoreType.DMA,
        ],
        compiler_params=pltpu.CompilerParams(disable_bounds_checks=True),
    )
    def gather_kernel(idx_hbm, table_hbm, out_hbm, idx_v, rows_v, s0, s1, s2, s3):
        sems = [s0, s1, s2, s3]
        wid = lax.axis_index("s") * _NC + lax.axis_index("c")
        bat_base = wid * bat_per_w

        pltpu.sync_copy(
            idx_hbm.at[pl.ds(bat_base * _SEQP, bat_per_w * _SEQP)], idx_v
        )

        def start_gather(g, b):
            pltpu.async_copy(
                table_hbm.at[idx_v.at[pl.ds(g * _SEQP, _SEQP)]],
                rows_v.at[b],
                sems[b],
            )

        def wait_gather(b):
            pltpu.make_async_copy(
                table_hbm.at[idx_v.at[pl.ds(0, _SEQP)]], rows_v.at[b], sems[b]
            ).wait()

        def start_write(g, b):
            pltpu.async_copy(
                rows_v.at[b],
                out_hbm.at[bat_base + g, pl.ds(0, _SEQP)],
                sems[b],
            )

        def wait_write(g, b):
            pltpu.make_async_copy(
                rows_v.at[b],
                out_hbm.at[bat_base + g, pl.ds(0, _SEQP)],
                sems[b],
            ).wait()

        for b in range(_NBUF):
            start_gather(b, b)

        def group(gi, carry):
            for b in range(_NBUF):
                g = gi * _NBUF + b
                wait_gather(b)
                start_write(g, b)
                wait_write(g, b)

                @pl.when(g + _NBUF < bat_per_w)
                def _():
                    start_gather(g + _NBUF, b)

            return carry

        lax.fori_loop(0, n_groups, group, 0)

    return gather_kernel


def kernel(indices, table):
    b, l = indices.shape
    idxp = jnp.pad(indices.astype(jnp.int32), ((0, 0), (0, _SEQP - l)))
    return _build(b, l)(idxp.reshape(-1), table)


# speedup vs baseline: 3.0995x; 3.0995x over previous
import functools

import jax
import jax.numpy as jnp
from jax import lax
from jax.experimental import pallas as pl
from jax.experimental.pallas import tpu as pltpu
from jax.experimental.pallas import tpu_sc as plsc

_EMBED = 512
_NC = 2
_NS = 16
_NW = _NC * _NS
_NBUF = 4
_SEQP = 56


@functools.cache
def _build(n_batch: int, seq: int):
    bat_per_w = n_batch // _NW
    n_groups = bat_per_w // _NBUF
    assert bat_per_w * _NW == n_batch
    assert n_groups * _NBUF == bat_per_w

    mesh = plsc.VectorSubcoreMesh(
        core_axis_name="c", subcore_axis_name="s", num_cores=_NC, num_subcores=_NS
    )

    @functools.partial(
        pl.kernel,
        mesh=mesh,
        out_type=jax.ShapeDtypeStruct((n_batch, seq, _EMBED), jnp.float32),
        scratch_types=[
            pltpu.VMEM((bat_per_w * _SEQP,), jnp.int32),
            pltpu.VMEM((_NBUF, _SEQP, _EMBED), jnp.float32),
            pltpu.SemaphoreType.DMA,
            pltpu.SemaphoreType.DMA,
            pltpu.SemaphoreType.DMA,
            pltpu.SemaphoreType.DMA,
        ],
        compiler_params=pltpu.CompilerParams(disable_bounds_checks=True),
    )
    def gather_kernel(idx_hbm, table_hbm, out_hbm, idx_v, rows_v, s0, s1, s2, s3):
        sems = [s0, s1, s2, s3]
        wid = lax.axis_index("s") * _NC + lax.axis_index("c")
        bat_base = wid * bat_per_w

        pltpu.sync_copy(
            idx_hbm.at[pl.ds(bat_base * _SEQP, bat_per_w * _SEQP)], idx_v
        )

        def start_gather(g, b):
            pltpu.async_copy(
                table_hbm.at[idx_v.at[pl.ds(g * _SEQP, _SEQP)]],
                rows_v.at[b],
                sems[b],
            )

        def wait_gather(b):
            pltpu.make_async_copy(
                table_hbm.at[idx_v.at[pl.ds(0, _SEQP)]], rows_v.at[b], sems[b]
            ).wait()

        def start_write(g, b):
            pltpu.async_copy(
                rows_v.at[b],
                out_hbm.at[bat_base + g, pl.ds(0, _SEQP)],
                sems[b],
            )

        def wait_write(g, b):
            pltpu.make_async_copy(
                rows_v.at[b],
                out_hbm.at[bat_base + g, pl.ds(0, _SEQP)],
                sems[b],
            ).wait()

        for b in range(_NBUF):
            start_gather(b, b)

        def group(gi, carry):
            for b in range(_NBUF):
                g = gi * _NBUF + b
                wait_gather(b)
                start_write(g, b)
                wait_write(g, b)

                @pl.when(g + _NBUF < bat_per_w)
                def _():
                    start_gather(g + _NBUF, b)

            return carry

        lax.fori_loop(0, n_groups, group, 0)

    return gather_kernel


def kernel(indices, table):
    b, l = indices.shape
    idxp = jnp.pad(indices.astype(jnp.int32), ((0, 0), (0, _SEQP - l)), mode="edge")
    return _build(b, l)(idxp.reshape(-1), table)


# speedup vs baseline: 3.1046x; 1.0016x over previous
import functools

import jax
import jax.numpy as jnp
from jax import lax
from jax.experimental import pallas as pl
from jax.experimental.pallas import tpu as pltpu
from jax.experimental.pallas import tpu_sc as plsc

_EMBED = 512
_NC = 2
_NS = 16
_NW = _NC * _NS
_NBUF = 2
_NB = 2
_SEQP = 56
_CROWS = _NB * _SEQP


@functools.cache
def _build(n_batch: int, seq: int):
    bat_per_w = n_batch // _NW
    n_chunks = bat_per_w // _NB
    n_groups = n_chunks // _NBUF
    assert bat_per_w * _NW == n_batch
    assert n_groups * _NBUF * _NB == bat_per_w

    mesh = plsc.VectorSubcoreMesh(
        core_axis_name="c", subcore_axis_name="s", num_cores=_NC, num_subcores=_NS
    )

    @functools.partial(
        pl.kernel,
        mesh=mesh,
        out_type=jax.ShapeDtypeStruct((n_batch, seq, _EMBED), jnp.float32),
        scratch_types=[
            pltpu.VMEM((bat_per_w * _SEQP,), jnp.int32),
            pltpu.VMEM((_NBUF, _CROWS, _EMBED), jnp.float32),
            pltpu.SemaphoreType.DMA,
            pltpu.SemaphoreType.DMA,
        ],
        compiler_params=pltpu.CompilerParams(disable_bounds_checks=True),
    )
    def gather_kernel(idx_hbm, table_hbm, out_hbm, idx_v, rows_v, s0, s1):
        sems = [s0, s1]
        wid = lax.axis_index("s") * _NC + lax.axis_index("c")
        bat_base = wid * bat_per_w

        pltpu.sync_copy(
            idx_hbm.at[pl.ds(bat_base * _SEQP, bat_per_w * _SEQP)], idx_v
        )

        def start_gather(g, b):
            pltpu.async_copy(
                table_hbm.at[idx_v.at[pl.ds(g * _CROWS, _CROWS)]],
                rows_v.at[b],
                sems[b],
            )

        def wait_gather(b):
            pltpu.make_async_copy(
                table_hbm.at[idx_v.at[pl.ds(0, _CROWS)]], rows_v.at[b], sems[b]
            ).wait()

        def start_write(g, b):
            for i in range(_NB):
                pltpu.async_copy(
                    rows_v.at[b, pl.ds(i * _SEQP, _SEQP)],
                    out_hbm.at[bat_base + g * _NB + i, pl.ds(0, _SEQP)],
                    sems[b],
                )

        def wait_write(g, b):
            for i in range(_NB):
                pltpu.make_async_copy(
                    rows_v.at[b, pl.ds(i * _SEQP, _SEQP)],
                    out_hbm.at[bat_base + g * _NB + i, pl.ds(0, _SEQP)],
                    sems[b],
                ).wait()

        for b in range(_NBUF):
            start_gather(b, b)

        def group(gi, carry):
            for b in range(_NBUF):
                g = gi * _NBUF + b
                wait_gather(b)
                start_write(g, b)
                wait_write(g, b)

                @pl.when(g + _NBUF < n_chunks)
                def _():
                    start_gather(g + _NBUF, b)

            return carry

        lax.fori_loop(0, n_groups, group, 0)

    return gather_kernel


def kernel(indices, table):
    b, l = indices.shape
    idxp = jnp.pad(indices.astype(jnp.int32), ((0, 0), (0, _SEQP - l)), mode="edge")
    return _build(b, l)(idxp.reshape(-1), table)
